# baseline (speedup 1.0000x reference)
import jax
import jax.numpy as jnp
from jax.experimental import pallas as pl
from jax.experimental.pallas import tpu as pltpu

LAMBDA_FAKE_W = 1.0
B = 256
D = 150528
RB = 8
NSTEPS = B // RB
NSLICE = 4
SLW = D // NSLICE


def _loss_kernel(y_ref, *refs):
    o_ref, acc_ref = refs[-2], refs[-1]
    in_refs = refs[:-2]
    step = pl.program_id(0)

    @pl.when(step == 0)
    def _init():
        acc_ref[0] = 0.0
        acc_ref[1] = 0.0
        acc_ref[2] = 0.0

    s = jnp.zeros((RB, 1), jnp.float32)
    for k in range(NSLICE):
        r_ref = in_refs[k]
        x_ref = in_refs[NSLICE + k]
        d = jnp.abs(r_ref[...] - x_ref[...])
        s = s + jnp.sum(d, axis=1, keepdims=True)
    yv = y_ref[...].astype(jnp.float32)
    acc_ref[0] += jnp.sum(s * yv)
    acc_ref[1] += jnp.sum(s)
    acc_ref[2] += jnp.sum(yv)

    @pl.when(step == NSTEPS - 1)
    def _finalize():
        n_real = acc_ref[2]
        n_fake = B - n_real
        sum_real = acc_ref[0]
        sum_fake = acc_ref[1] - sum_real
        loss_real = jnp.where(n_real > 0, sum_real / (n_real * D), 0.0)
        loss_fake = jnp.where(n_fake > 0, sum_fake / (n_fake * D), 0.0)
        o_ref[...] = (loss_real - LAMBDA_FAKE_W * loss_fake).reshape(1, 1)


def kernel(recons, x, y):
    r2 = recons.reshape(B, D)
    x2 = x.reshape(B, D)
    y2 = y.astype(jnp.float32).reshape(B, 1)

    operands = [r2] * NSLICE + [x2] * NSLICE

    def _mk_spec(k):
        return pl.BlockSpec((RB, SLW), lambda i, _k=k: (i, _k))

    big_specs = [_mk_spec(k) for k in range(NSLICE)] * 2
    out = pl.pallas_call(
        _loss_kernel,
        grid=(NSTEPS,),
        in_specs=[pl.BlockSpec((RB, 1), lambda i: (i, 0))] + big_specs,
        out_specs=pl.BlockSpec((1, 1), lambda i: (0, 0)),
        out_shape=jax.ShapeDtypeStruct((1, 1), jnp.float32),
        scratch_shapes=[pltpu.SMEM((3,), jnp.float32)],
        compiler_params=pltpu.CompilerParams(
            dimension_semantics=("arbitrary",),
        ),
    )(y2, *operands)
    return out.reshape(())
